# baseline (speedup 1.0000x reference)
import jax
import jax.numpy as jnp
from jax.experimental import pallas as pl
from jax.experimental.pallas import tpu as pltpu

N = 10000
F = 128
TM = 256
H = TM // 2


def _body(ka_ref, kb_ref, x_ref, b_ref, o_ref):
    x = x_ref[...]
    o_ref[0:H, :] = (
        jnp.dot(ka_ref[...], x, preferred_element_type=jnp.float32)
        + b_ref[0:H, :]
    )
    o_ref[H:TM, :] = (
        jnp.dot(kb_ref[...], x, preferred_element_type=jnp.float32)
        + b_ref[H:TM, :]
    )


def kernel(input_, kernel, bias):
    x = input_.reshape(N, F)
    out = pl.pallas_call(
        _body,
        grid=(pl.cdiv(N, TM),),
        in_specs=[
            pl.BlockSpec((H, N), lambda i: (2 * i, 0)),
            pl.BlockSpec((H, N), lambda i: (jnp.minimum(2 * i + 1, N // H), 0)),
            pl.BlockSpec((N, F), lambda i: (0, 0)),
            pl.BlockSpec((TM, F), lambda i: (i, 0)),
        ],
        out_specs=pl.BlockSpec((TM, F), lambda i: (i, 0)),
        out_shape=jax.ShapeDtypeStruct((N, F), jnp.float32),
        compiler_params=pltpu.CompilerParams(
            dimension_semantics=("arbitrary",),
        ),
    )(kernel, kernel, x, bias)
    return out.reshape(1, N, F)


# speedup vs baseline: 1.0044x; 1.0044x over previous
import jax
import jax.numpy as jnp
from jax.experimental import pallas as pl
from jax.experimental.pallas import tpu as pltpu

N = 10000
F = 128
TM = 256


def _body(k_ref, x_ref, b_ref, o_ref):
    o_ref[...] = (
        jnp.dot(k_ref[...], x_ref[...], preferred_element_type=jnp.float32)
        + b_ref[...]
    )


def kernel(input_, kernel, bias):
    x = input_.reshape(N, F)
    out = pl.pallas_call(
        _body,
        grid=(pl.cdiv(N, TM),),
        in_specs=[
            pl.BlockSpec((TM, N), lambda i: (i, 0)),
            pl.BlockSpec((N, F), lambda i: (0, 0)),
            pl.BlockSpec((TM, F), lambda i: (i, 0)),
        ],
        out_specs=pl.BlockSpec((TM, F), lambda i: (i, 0)),
        out_shape=jax.ShapeDtypeStruct((N, F), jnp.float32),
        compiler_params=pltpu.CompilerParams(
            dimension_semantics=("parallel",),
        ),
    )(kernel, x, bias)
    return out.reshape(1, N, F)


# speedup vs baseline: 1.0196x; 1.0152x over previous
import jax
import jax.numpy as jnp
from jax.experimental import pallas as pl
from jax.experimental.pallas import tpu as pltpu

N = 10000
F = 128
TM = 256
G = pl.cdiv(N, TM)


def _body(k_ref, x_ref, b_ref, o_ref):
    o_ref[...] = (
        jnp.dot(k_ref[...], x_ref[...], preferred_element_type=jnp.float32)
        + b_ref[...]
    )


def kernel(input_, kernel, bias):
    x = input_.reshape(N, F)
    out = pl.pallas_call(
        _body,
        grid=(G,),
        in_specs=[
            pl.BlockSpec((TM, N), lambda i: (G - 1 - i, 0)),
            pl.BlockSpec((N, F), lambda i: (0, 0)),
            pl.BlockSpec((TM, F), lambda i: (G - 1 - i, 0)),
        ],
        out_specs=pl.BlockSpec((TM, F), lambda i: (G - 1 - i, 0)),
        out_shape=jax.ShapeDtypeStruct((N, F), jnp.float32),
        compiler_params=pltpu.CompilerParams(
            dimension_semantics=("parallel",),
        ),
    )(kernel, x, bias)
    return out.reshape(1, N, F)
